# Initial kernel scaffold; baseline (speedup 1.0000x reference)
import jax
import jax.numpy as jnp
from jax.experimental import pallas as pl


def kernel(x, embed_w, pos_w):
    raise NotImplementedError("write your pallas kernel here")


# speedup vs baseline: 1.9227x; 1.9227x over previous
import functools

import jax
import jax.numpy as jnp
from jax import lax
from jax.experimental import pallas as pl
from jax.experimental.pallas import tpu as pltpu
from jax.experimental.pallas import tpu_sc as plsc

VOCAB = 1000000
EMB = 64
MAXLEN = 200
BATCH = 4096
SEQ = 200

NC = 2
NS = 16
NW = NC * NS
N = BATCH * SEQ
ROWS_PER_W = N // NW
GB = 128
GATHERS = ROWS_PER_W // GB
LANES = 16
ECH = EMB // LANES


def _body(idx_hbm, table_hbm, pos_hbm, out_hbm, idx_v, rows_v, pos_v, sem):
    cid = lax.axis_index("c")
    sid = lax.axis_index("s")
    wid = sid * NC + cid

    pltpu.sync_copy(pos_hbm, pos_v)
    inv_scale = jnp.float32(1.0 / float(EMB) ** 0.5)

    def scale_body(i, carry):
        for c in range(ECH):
            sl = pl.ds(c * LANES, LANES)
            pos_v[i, sl] = pos_v[i, sl] * inv_scale
        return carry

    lax.fori_loop(0, MAXLEN, scale_body, 0)

    pltpu.sync_copy(idx_hbm.at[pl.ds(wid * GATHERS, GATHERS)], idx_v)

    def g_body(j, carry):
        pltpu.async_copy(table_hbm.at[idx_v.at[j]], rows_v, sem).wait()

        def row_body(i, c2):
            s = lax.rem(j * GB + i, MAXLEN)
            for c in range(ECH):
                sl = pl.ds(c * LANES, LANES)
                rows_v[i, sl] = rows_v[i, sl] + pos_v[s, sl]
            return c2

        lax.fori_loop(0, GB, row_body, 0)
        pltpu.sync_copy(rows_v, out_hbm.at[pl.ds(wid * ROWS_PER_W + j * GB, GB)])
        return carry

    lax.fori_loop(0, GATHERS, g_body, 0)


@jax.jit
def _sc_embed(xf, embed_w, pos_w):
    mesh = plsc.VectorSubcoreMesh(core_axis_name="c", subcore_axis_name="s")
    f = functools.partial(
        pl.kernel,
        mesh=mesh,
        out_type=jax.ShapeDtypeStruct((N, EMB), jnp.float32),
        scratch_types=[
            pltpu.VMEM((GATHERS, GB), jnp.int32),
            pltpu.VMEM((GB, EMB), jnp.float32),
            pltpu.VMEM((MAXLEN, EMB), jnp.float32),
            pltpu.SemaphoreType.DMA,
        ],
        compiler_params=pltpu.CompilerParams(use_tc_tiling_on_sc=False),
    )(_body)
    return f(xf, embed_w, pos_w)


def kernel(x, embed_w, pos_w):
    xf = x.reshape(NW * GATHERS, GB)
    out = _sc_embed(xf, embed_w, pos_w)
    return out.reshape(BATCH, SEQ, EMB)


# speedup vs baseline: 2.8446x; 1.4794x over previous
import functools

import jax
import jax.numpy as jnp
from jax import lax
from jax.experimental import pallas as pl
from jax.experimental.pallas import tpu as pltpu
from jax.experimental.pallas import tpu_sc as plsc

VOCAB = 1000000
EMB = 64
MAXLEN = 200
BATCH = 4096
SEQ = 200

NC = 2
NS = 16
NW = NC * NS
N = BATCH * SEQ
GB = 128
UNITS = N // GB
UNITS_PER_W = UNITS // NW
LANES = 16
ECH = EMB // LANES
ET = EMB // 8
BB = BATCH // GB
NBG = 3
NBT = 2


def _body(idx_hbm, table_hbm, pos_hbm, out_hbm, idx_v, gbuf, tbuf, pos_v,
          gsem, wsem):
    cid = lax.axis_index("c")
    sid = lax.axis_index("s")
    wid = sid * NC + cid
    u0 = wid * UNITS_PER_W

    pltpu.sync_copy(pos_hbm, pos_v)
    inv_scale = jnp.float32(1.0 / float(EMB) ** 0.5)

    @plsc.parallel_loop(0, MAXLEN, 1, unroll=4)
    def _scale(i):
        for c in range(ECH):
            sl = pl.ds(c * LANES, LANES)
            pos_v[i, sl] = pos_v[i, sl] * inv_scale

    pltpu.sync_copy(idx_hbm.at[pl.ds(u0, UNITS_PER_W)], idx_v)

    @plsc.parallel_loop(0, UNITS_PER_W, 1, unroll=2)
    def _dbl(j):
        for c in range(GB // LANES):
            sl = pl.ds(c * LANES, LANES)
            idx_v[j, sl] = idx_v[j, sl] * 2

    def _gather(j, bg, start):
        mk = pltpu.async_copy if start else pltpu.make_async_copy
        return mk(table_hbm.at[idx_v.at[j]], gbuf.at[bg], gsem.at[bg])

    _gather(0, 0, True)
    _gather(1, 1, True)

    def u_body(j, carry):
        u = u0 + j
        s = lax.div(u, BB)
        bb = lax.rem(u, BB)
        bg = lax.rem(j, NBG)
        bt = lax.rem(j, NBT)

        _gather(j, bg, False).wait()

        @pl.when(j >= NBT)
        def _():
            for e8 in range(ET):
                pltpu.make_async_copy(
                    tbuf.at[bt, e8], out_hbm.at[s, e8, bb], wsem.at[bt]
                ).wait()

        lane_ids = lax.iota(jnp.int32, LANES)

        sv = jnp.full((LANES,), s, jnp.int32)

        @plsc.parallel_loop(0, EMB, 1, unroll=2)
        def _tr(e):
            e8 = lax.div(e, 8)
            ei = lax.rem(e, 8)
            bgv = jnp.full((LANES,), bg, jnp.int32)
            ev = jnp.full((LANES,), e, jnp.int32)
            pe = plsc.load_gather(pos_v, [sv, ev])
            for bc in range(GB // LANES):
                rows = lane_ids + (bc * LANES)
                vals = plsc.load_gather(gbuf, [bgv, rows, ev])
                tbuf[bt, e8, ei, pl.ds(bc * LANES, LANES)] = vals + pe

        for e8 in range(ET):
            pltpu.async_copy(tbuf.at[bt, e8], out_hbm.at[s, e8, bb],
                             wsem.at[bt])

        @pl.when(j + 2 < UNITS_PER_W)
        def _():
            _gather(j + 2, lax.rem(j + 2, NBG), True)

        return carry

    lax.fori_loop(0, UNITS_PER_W, u_body, 0)

    for k in range(NBT):
        j = UNITS_PER_W - NBT + k
        u = u0 + j
        s = lax.div(jnp.int32(u), BB)
        bb = lax.rem(jnp.int32(u), BB)
        for e8 in range(ET):
            pltpu.make_async_copy(
                tbuf.at[j % NBT, e8], out_hbm.at[s, e8, bb],
                wsem.at[j % NBT]
            ).wait()


@jax.jit
def _sc_embed(xf, table2, pos_w):
    mesh = plsc.VectorSubcoreMesh(core_axis_name="c", subcore_axis_name="s")
    f = functools.partial(
        pl.kernel,
        mesh=mesh,
        out_type=jax.ShapeDtypeStruct((SEQ, ET, BB, 8, GB), jnp.float32),
        scratch_types=[
            pltpu.VMEM((UNITS_PER_W, GB), jnp.int32),
            pltpu.VMEM((NBG, GB, EMB), jnp.float32),
            pltpu.VMEM((NBT, ET, 8, GB), jnp.float32),
            pltpu.VMEM((MAXLEN, EMB), jnp.float32),
            pltpu.SemaphoreType.DMA((NBG,)),
            pltpu.SemaphoreType.DMA((NBT,)),
        ],
        compiler_params=pltpu.CompilerParams(use_tc_tiling_on_sc=False, needs_layout_passes=False),
    )(_body)
    return f(xf, table2, pos_w)


def kernel(x, embed_w, pos_w):
    xf = jnp.transpose(x).reshape(UNITS, GB)
    table2 = jnp.pad(embed_w, ((0, 0), (0, 64))).reshape(2 * VOCAB, EMB)
    w = _sc_embed(xf, table2, pos_w)
    return w.transpose(2, 4, 0, 1, 3).reshape(BATCH, SEQ, EMB)
